# baseline (speedup 1.0000x reference)
import functools

import jax
import jax.numpy as jnp
from jax import lax
from jax.experimental import pallas as pl
from jax.experimental.pallas import tpu as pltpu
from jax.experimental.pallas import tpu_sc as plsc

_LANES = 16
_JH = 16
_ACHUNK = 128
_BNB = 2


def _iota16():
    return lax.broadcasted_iota(jnp.int32, (_LANES,), 0)


def _splat(x):
    return jnp.full((_LANES,), x, jnp.int32)


def _transpose_a(src, dst, nrows):
    row16 = _iota16()

    @pl.loop(0, nrows // 8)
    def _o(o):
        for r in range(8):
            v = plsc.load_gather(src, [row16, _splat(o * 8 + r)])
            plsc.store_scatter(dst, [_splat(o), r * 16 + row16], v)


def _transpose_b(src, dst):
    row16 = _iota16()

    @pl.loop(0, _JH)
    def _j(j):
        for k in range(8):
            p = k * 256 + row16 * _JH + j
            v = plsc.load_gather(src, [p >> 7, p & 127])
            dst[j, pl.ds(k * 16, 16)] = v


def _body(n_full, tail_w, tail_off, a_per_tile, b_units, s_len,
          wT, idxT, outT, wlin,
          a_in, a_out, a_tin, a_tout, aidx, idx_v, gbufs, tbufs,
          arsems, awsems, gsems, ssems):
    c = lax.axis_index("c")
    s_id = lax.axis_index("s")
    jbase = c * _JH
    row16 = _iota16()
    wmine = wlin.at[c]

    def a_chunk(m):
        ch = s_id * a_per_tile + m
        return ch - n_full * (ch >= n_full)

    def a_rstart(m, p):
        ch = a_chunk(m)
        pltpu.async_copy(
            wT.at[pl.ds(jbase, _JH), pl.ds(ch * _ACHUNK, _ACHUNK)],
            a_in[p], arsems[p])

    def a_rwait(p):
        pltpu.make_async_copy(
            wT.at[pl.ds(0, _JH), pl.ds(0, _ACHUNK)], a_in[p], arsems[p]).wait()

    def a_wstart(m, p):
        ch = a_chunk(m)
        for k in range(8):
            aidx[p][pl.ds(k * 16, 16)] = ch * _ACHUNK + k * 16 + row16
        pltpu.async_copy(a_out[p], wmine.at[aidx[p]], awsems[p])

    def a_wwait(p):
        pltpu.make_async_copy(a_out[p], wmine.at[aidx[p]], awsems[p]).wait()

    a_rstart(0, 0)
    a_rstart(1, 1)

    @pl.loop(0, (a_per_tile - 1) // 2)
    def _aloop(o):
        for p in range(2):
            m = o * 2 + p
            a_rwait(p)

            @pl.when(m >= 2)
            def _():
                a_wwait(p)

            _transpose_a(a_in[p], a_out[p], _ACHUNK)
            a_wstart(m, p)

            @pl.when(m + 2 < a_per_tile)
            def _():
                a_rstart(m + 2, p)

    a_rwait(0)
    a_wwait(0)
    _transpose_a(a_in[0], a_out[0], _ACHUNK)
    a_wstart(a_per_tile - 1, 0)
    a_wwait(0)
    a_wwait(1)

    @pl.when(s_id == 15)
    def _tail():
        pltpu.sync_copy(
            wT.at[pl.ds(jbase, _JH), pl.ds(tail_off, tail_w)], a_tin)
        _transpose_a(a_tin, a_tout, tail_w)
        for k in range(tail_w // 16):
            aidx[0][pl.ds(k * 16, 16)] = tail_off + k * 16 + row16
        pltpu.async_copy(
            a_tout, wmine.at[aidx[0].at[pl.ds(0, tail_w)]], awsems[0])
        pltpu.make_async_copy(
            a_tout, wmine.at[aidx[0].at[pl.ds(0, tail_w)]], awsems[0]).wait()

    plsc.subcore_barrier()

    for ph in range(2):
        bcol = s_id * s_len + ph * (s_len // 2)
        pltpu.sync_copy(idxT.at[:, pl.ds(bcol, s_len // 2)], idx_v)

        def g_start(u, nb):
            t = u // 4
            l = u % 4
            pltpu.async_copy(
                wmine.at[idx_v.at[t, pl.ds(l * 128, 128)]], gbufs[nb],
                gsems[nb])

        def g_wait(nb):
            pltpu.make_async_copy(
                wmine.at[idx_v.at[0, pl.ds(0, 128)]], gbufs[nb],
                gsems[nb]).wait()

        def s_start(u, nb):
            t = u // 4
            l = u % 4
            pltpu.async_copy(
                tbufs[nb],
                outT.at[t, pl.ds(jbase, _JH), pl.ds(bcol + l * 128, 128)],
                ssems[nb])

        def s_wait(nb):
            pltpu.make_async_copy(
                tbufs[nb], outT.at[0, pl.ds(0, _JH), pl.ds(0, 128)],
                ssems[nb]).wait()

        for nb in range(_BNB):
            g_start(nb, nb)

        @pl.loop(0, b_units // 2 // _BNB)
        def _bloop(o):
            for nb in range(_BNB):
                u = o * _BNB + nb
                g_wait(nb)

                @pl.when(u >= _BNB)
                def _():
                    s_wait(nb)

                _transpose_b(gbufs[nb], tbufs[nb])
                s_start(u, nb)

                @pl.when(u + _BNB < b_units // 2)
                def _():
                    g_start(u + _BNB, nb)

        for nb in range(_BNB):
            s_wait(nb)


def kernel(input, weight):
    n, s = input.shape
    num_rows, dim = weight.shape
    idxT = input.astype(jnp.int32).T
    wT = weight.T

    info = plsc.get_sparse_core_info()
    num_tiles = info.num_subcores

    n_full = num_rows // _ACHUNK
    tail_w = num_rows - n_full * _ACHUNK
    tail_off = n_full * _ACHUNK
    a_per_tile = -(-n_full // num_tiles)
    assert a_per_tile % 2 == 1

    s_len = n // num_tiles
    b_units = s * (s_len // 128)

    mesh = plsc.VectorSubcoreMesh(core_axis_name="c", subcore_axis_name="s")
    fused = pl.kernel(
        functools.partial(_body, n_full, tail_w, tail_off, a_per_tile,
                          b_units, s_len),
        out_type=jax.ShapeDtypeStruct((s, dim, n), jnp.float32),
        mesh=mesh,
        scratch_types=[
            pltpu.HBM((2, num_rows, _JH), jnp.float32),
            [pltpu.VMEM((_JH, _ACHUNK), jnp.float32) for _ in range(2)],
            [pltpu.VMEM((_ACHUNK, _JH), jnp.float32) for _ in range(2)],
            pltpu.VMEM((_JH, 64), jnp.float32),
            pltpu.VMEM((64, _JH), jnp.float32),
            [pltpu.VMEM((_ACHUNK,), jnp.int32) for _ in range(2)],
            pltpu.VMEM((s, 512), jnp.int32),
            [pltpu.VMEM((128, _JH), jnp.float32) for _ in range(_BNB)],
            [pltpu.VMEM((_JH, 128), jnp.float32) for _ in range(_BNB)],
            [pltpu.SemaphoreType.DMA for _ in range(2)],
            [pltpu.SemaphoreType.DMA for _ in range(2)],
            [pltpu.SemaphoreType.DMA for _ in range(_BNB)],
            [pltpu.SemaphoreType.DMA for _ in range(_BNB)],
        ],
        compiler_params=pltpu.CompilerParams(
            use_tc_tiling_on_sc=True, needs_layout_passes=False),
    )
    outT = fused(wT, idxT)
    return outT.transpose(2, 0, 1)


# speedup vs baseline: 1.1345x; 1.1345x over previous
import functools

import jax
import jax.numpy as jnp
from jax import lax
from jax.experimental import pallas as pl
from jax.experimental.pallas import tpu as pltpu
from jax.experimental.pallas import tpu_sc as plsc

_NBUF = 8


def _gather_body(rows_per_w, num_cores,
                 table_hbm, idx_hbm, out_hbm, idx_v, rows, gsems, ssems):
    wid = lax.axis_index("s") * num_cores + lax.axis_index("c")
    base = wid * rows_per_w

    pltpu.sync_copy(idx_hbm.at[pl.ds(base, rows_per_w), :], idx_v)

    def gather_start(c, b):
        pltpu.async_copy(table_hbm.at[idx_v.at[c]], rows[b], gsems[b])

    def gather_wait(b):
        pltpu.make_async_copy(table_hbm.at[idx_v.at[0]], rows[b],
                              gsems[b]).wait()

    def store_start(c, b):
        pltpu.async_copy(rows[b], out_hbm.at[base + c], ssems[b])

    def store_wait(b):
        pltpu.make_async_copy(rows[b], out_hbm.at[0], ssems[b]).wait()

    for b in range(_NBUF):
        gather_start(b, b)

    @pl.loop(0, rows_per_w // _NBUF)
    def _outer(o):
        for b in range(_NBUF):
            c = o * _NBUF + b
            gather_wait(b)
            store_start(c, b)
            cn = c + _NBUF

            @pl.when(cn < rows_per_w)
            def _():
                store_wait(b)
                gather_start(cn, b)

    for b in range(_NBUF):
        store_wait(b)


def kernel(input, weight):
    n, s = input.shape
    num_rows, dim = weight.shape
    idx = input.astype(jnp.int32)

    info = plsc.get_sparse_core_info()
    num_workers = info.num_cores * info.num_subcores
    rows_per_w = n // num_workers

    mesh = plsc.VectorSubcoreMesh(core_axis_name="c", subcore_axis_name="s")
    gather = pl.kernel(
        functools.partial(_gather_body, rows_per_w, info.num_cores),
        out_type=jax.ShapeDtypeStruct((n, s, dim), jnp.float32),
        mesh=mesh,
        scratch_types=[
            pltpu.VMEM((rows_per_w, s), jnp.int32),
            [pltpu.VMEM((s, dim), jnp.float32) for _ in range(_NBUF)],
            [pltpu.SemaphoreType.DMA for _ in range(_NBUF)],
            [pltpu.SemaphoreType.DMA for _ in range(_NBUF)],
        ],
        compiler_params=pltpu.CompilerParams(use_tc_tiling_on_sc=False),
    )
    return gather(weight, idx)
